# speedup vs baseline: 1.8918x; 1.8918x over previous
import functools

import jax
import jax.numpy as jnp
from jax import lax
from jax.experimental import pallas as pl
from jax.experimental.pallas import tpu as pltpu
from jax.experimental.pallas import tpu_sc as plsc

N_FINE = 500000
N_COARSE = 100000
DIM = 128
LANES = 16

C = 80
N_CHUNKS = N_FINE // C
NUM_WORKERS = 32
MAX_CHUNKS_PER_WORKER = -(-N_CHUNKS // NUM_WORKERS)


def _sc_kernel(x_hbm, up_hbm, inv_hbm, out_hbm, idx_v, rows_v, up_v, sem):
    nc = 2
    wid = lax.axis_index("s") * nc + lax.axis_index("c")

    def chunk_body(i, _):
        chunk = wid + i * NUM_WORKERS

        @pl.when(chunk < N_CHUNKS)
        def _():
            start = chunk * C
            pltpu.sync_copy(inv_hbm.at[pl.ds(start, C)], idx_v)
            gather = pltpu.async_copy(x_hbm.at[idx_v], rows_v, sem)
            pltpu.sync_copy(up_hbm.at[pl.ds(start, C)], up_v)
            gather.wait()

            def add_row(r, _):
                for l in range(DIM // LANES):
                    sl = pl.ds(l * LANES, LANES)
                    rows_v[r, sl] += up_v[r, sl]
                return 0

            lax.fori_loop(0, C, add_row, 0)
            pltpu.sync_copy(rows_v, out_hbm.at[pl.ds(start, C)])

        return 0

    lax.fori_loop(0, MAX_CHUNKS_PER_WORKER, chunk_body, 0)


@jax.jit
def _run(x_features, up_x_features, unq_inv):
    mesh = plsc.VectorSubcoreMesh(core_axis_name="c", subcore_axis_name="s")
    return pl.kernel(
        _sc_kernel,
        mesh=mesh,
        out_type=jax.ShapeDtypeStruct((N_FINE, DIM), jnp.float32),
        scratch_types=[
            pltpu.VMEM((C,), jnp.int32),
            pltpu.VMEM((C, DIM), jnp.float32),
            pltpu.VMEM((C, DIM), jnp.float32),
            pltpu.SemaphoreType.DMA,
        ],
    )(x_features, up_x_features, unq_inv)


def kernel(x_features, up_x_features, unq_inv):
    return _run(x_features, up_x_features, unq_inv.astype(jnp.int32))


# speedup vs baseline: 3.2395x; 1.7124x over previous
import functools

import jax
import jax.numpy as jnp
from jax import lax
from jax.experimental import pallas as pl
from jax.experimental.pallas import tpu as pltpu
from jax.experimental.pallas import tpu_sc as plsc

N_FINE = 500000
N_COARSE = 100000
DIM = 128
LANES = 16

C = 80
N_CHUNKS = N_FINE // C
NUM_WORKERS = 32
J_MAX = -(-N_CHUNKS // NUM_WORKERS)
J_MAX += J_MAX % 2


def _sc_kernel(x_hbm, up_hbm, inv_hbm, out_hbm,
               idx0, idx1, rows0, rows1, upb0, upb1,
               sg0, sg1, su0, su1, so0, so1):
    idxb = (idx0, idx1)
    rows = (rows0, rows1)
    upb = (upb0, upb1)
    sg = (sg0, sg1)
    su = (su0, su1)
    so = (so0, so1)
    wid = lax.axis_index("s") * 2 + lax.axis_index("c")

    def exists(j):
        return wid + j * NUM_WORKERS < N_CHUNKS

    def start_of(j):
        return (wid + j * NUM_WORKERS) * C

    def issue_loads(j, b):
        @pl.when(exists(j))
        def _():
            s = start_of(j)
            pltpu.sync_copy(inv_hbm.at[pl.ds(s, C)], idxb[b])
            pltpu.async_copy(x_hbm.at[idxb[b]], rows[b], sg[b])
            pltpu.async_copy(up_hbm.at[pl.ds(s, C)], upb[b], su[b])

    def wait_rows_free(j, b):
        @pl.when(exists(j) & (j >= 2))
        def _():
            pltpu.make_async_copy(rows[b], out_hbm.at[pl.ds(0, C)], so[b]).wait()

    def compute_write(j, b):
        @pl.when(exists(j))
        def _():
            pltpu.make_async_copy(x_hbm.at[idxb[b]], rows[b], sg[b]).wait()
            pltpu.make_async_copy(up_hbm.at[pl.ds(0, C)], upb[b], su[b]).wait()

            @plsc.parallel_loop(0, C, step=1, unroll=4)
            def _add_row(r):
                for l in range(DIM // LANES):
                    sl = pl.ds(l * LANES, LANES)
                    plsc.addupdate(rows[b].at[r, sl], upb[b][r, sl])

            pltpu.async_copy(rows[b], out_hbm.at[pl.ds(start_of(j), C)], so[b])

    issue_loads(0, 0)

    def pair_body(i, _):
        for bb in range(2):
            j = 1 + 2 * i + bb
            b = (1 + bb) % 2
            wait_rows_free(j, b)
            issue_loads(j, b)
            compute_write(j - 1, 1 - b)
        return 0

    lax.fori_loop(0, J_MAX // 2, pair_body, 0)

    pltpu.make_async_copy(rows[0], out_hbm.at[pl.ds(0, C)], so[0]).wait()
    pltpu.make_async_copy(rows[1], out_hbm.at[pl.ds(0, C)], so[1]).wait()


@jax.jit
def _run(x_features, up_x_features, unq_inv):
    mesh = plsc.VectorSubcoreMesh(core_axis_name="c", subcore_axis_name="s")
    return pl.kernel(
        _sc_kernel,
        mesh=mesh,
        out_type=jax.ShapeDtypeStruct((N_FINE, DIM), jnp.float32),
        scratch_types=[
            pltpu.VMEM((C,), jnp.int32),
            pltpu.VMEM((C,), jnp.int32),
            pltpu.VMEM((C, DIM), jnp.float32),
            pltpu.VMEM((C, DIM), jnp.float32),
            pltpu.VMEM((C, DIM), jnp.float32),
            pltpu.VMEM((C, DIM), jnp.float32),
            pltpu.SemaphoreType.DMA,
            pltpu.SemaphoreType.DMA,
            pltpu.SemaphoreType.DMA,
            pltpu.SemaphoreType.DMA,
            pltpu.SemaphoreType.DMA,
            pltpu.SemaphoreType.DMA,
        ],
    )(x_features, up_x_features, unq_inv)


def kernel(x_features, up_x_features, unq_inv):
    return _run(x_features, up_x_features, unq_inv.astype(jnp.int32))


# speedup vs baseline: 4.0594x; 1.2531x over previous
import functools

import jax
import jax.numpy as jnp
from jax import lax
from jax.experimental import pallas as pl
from jax.experimental.pallas import tpu as pltpu
from jax.experimental.pallas import tpu_sc as plsc

N_FINE = 500000
N_COARSE = 100000
DIM = 128
LANES = 16

C = 80
N_CHUNKS = N_FINE // C
NUM_WORKERS = 32
NBUF = 3
J_MAX = -(-N_CHUNKS // NUM_WORKERS)
J_MAX = -(-(J_MAX + 2) // NBUF) * NBUF


def _sc_kernel(x_hbm, up_hbm, inv_hbm, out_hbm,
               idx0, idx1, idx2, rows0, rows1, rows2, upb0, upb1, upb2,
               si0, si1, si2, sg0, sg1, sg2, su0, su1, su2, so0, so1, so2):
    idxb = (idx0, idx1, idx2)
    rows = (rows0, rows1, rows2)
    upb = (upb0, upb1, upb2)
    si = (si0, si1, si2)
    sg = (sg0, sg1, sg2)
    su = (su0, su1, su2)
    so = (so0, so1, so2)
    wid = lax.axis_index("s") * 2 + lax.axis_index("c")

    def exists(j):
        return wid + j * NUM_WORKERS < N_CHUNKS

    def start_of(j):
        return (wid + j * NUM_WORKERS) * C

    def prefetch_idx(j, b):
        @pl.when(exists(j))
        def _():
            pltpu.async_copy(inv_hbm.at[pl.ds(start_of(j), C)], idxb[b], si[b])

    def issue_loads(j, b):
        @pl.when(exists(j))
        def _():
            @pl.when(j >= NBUF)
            def _():
                pltpu.make_async_copy(
                    rows[b], out_hbm.at[pl.ds(0, C)], so[b]).wait()

            pltpu.make_async_copy(
                inv_hbm.at[pl.ds(0, C)], idxb[b], si[b]).wait()
            pltpu.async_copy(x_hbm.at[idxb[b]], rows[b], sg[b])
            pltpu.async_copy(up_hbm.at[pl.ds(start_of(j), C)], upb[b], su[b])

    def compute_write(j, b):
        @pl.when(exists(j))
        def _():
            pltpu.make_async_copy(x_hbm.at[idxb[b]], rows[b], sg[b]).wait()
            pltpu.make_async_copy(up_hbm.at[pl.ds(0, C)], upb[b], su[b]).wait()

            @plsc.parallel_loop(0, C, step=1, unroll=4)
            def _add_row(r):
                for l in range(DIM // LANES):
                    sl = pl.ds(l * LANES, LANES)
                    plsc.addupdate(rows[b].at[r, sl], upb[b][r, sl])

            pltpu.async_copy(rows[b], out_hbm.at[pl.ds(start_of(j), C)], so[b])

    prefetch_idx(0, 0)
    prefetch_idx(1, 1)
    issue_loads(0, 0)

    def triple_body(i, _):
        for bb in range(NBUF):
            j = 1 + NBUF * i + bb
            b = (1 + bb) % NBUF
            issue_loads(j, b)
            prefetch_idx(j + 1, (b + 1) % NBUF)
            compute_write(j - 1, bb)
        return 0

    lax.fori_loop(0, J_MAX // NBUF, triple_body, 0)

    for b in range(NBUF):
        pltpu.make_async_copy(rows[b], out_hbm.at[pl.ds(0, C)], so[b]).wait()


@jax.jit
def _run(x_features, up_x_features, unq_inv):
    mesh = plsc.VectorSubcoreMesh(core_axis_name="c", subcore_axis_name="s")
    return pl.kernel(
        _sc_kernel,
        mesh=mesh,
        out_type=jax.ShapeDtypeStruct((N_FINE, DIM), jnp.float32),
        scratch_types=(
            [pltpu.VMEM((C,), jnp.int32)] * NBUF
            + [pltpu.VMEM((C, DIM), jnp.float32)] * (2 * NBUF)
            + [pltpu.SemaphoreType.DMA] * (4 * NBUF)
        ),
    )(x_features, up_x_features, unq_inv)


def kernel(x_features, up_x_features, unq_inv):
    return _run(x_features, up_x_features, unq_inv.astype(jnp.int32))


# speedup vs baseline: 4.1228x; 1.0156x over previous
import functools

import jax
import jax.numpy as jnp
from jax import lax
from jax.experimental import pallas as pl
from jax.experimental.pallas import tpu as pltpu
from jax.experimental.pallas import tpu_sc as plsc

N_FINE = 500000
N_COARSE = 100000
DIM = 128
LANES = 16

C = 128
N_CHUNKS = N_FINE // C
TAIL = N_FINE - N_CHUNKS * C
TAIL_START = N_CHUNKS * C
NUM_WORKERS = 32
NBUF = 3
J_MAX = -(-N_CHUNKS // NUM_WORKERS)
J_MAX = -(-(J_MAX + 2) // NBUF) * NBUF


def _sc_kernel(x_hbm, up_hbm, inv_hbm, out_hbm,
               idx0, idx1, idx2, rows0, rows1, rows2, upb0, upb1, upb2,
               si0, si1, si2, sg0, sg1, sg2, su0, su1, su2, so0, so1, so2):
    idxb = (idx0, idx1, idx2)
    rows = (rows0, rows1, rows2)
    upb = (upb0, upb1, upb2)
    si = (si0, si1, si2)
    sg = (sg0, sg1, sg2)
    su = (su0, su1, su2)
    so = (so0, so1, so2)
    wid = lax.axis_index("s") * 2 + lax.axis_index("c")

    def exists(j):
        return wid + j * NUM_WORKERS < N_CHUNKS

    def start_of(j):
        return (wid + j * NUM_WORKERS) * C

    def prefetch_idx(j, b):
        @pl.when(exists(j))
        def _():
            pltpu.async_copy(inv_hbm.at[pl.ds(start_of(j), C)], idxb[b], si[b])

    def issue_loads(j, b):
        @pl.when(exists(j))
        def _():
            @pl.when(j >= NBUF)
            def _():
                pltpu.make_async_copy(
                    rows[b], out_hbm.at[pl.ds(0, C)], so[b]).wait()

            pltpu.make_async_copy(
                inv_hbm.at[pl.ds(0, C)], idxb[b], si[b]).wait()
            pltpu.async_copy(x_hbm.at[idxb[b]], rows[b], sg[b])
            pltpu.async_copy(up_hbm.at[pl.ds(start_of(j), C)], upb[b], su[b])

    def compute_write(j, b):
        @pl.when(exists(j))
        def _():
            pltpu.make_async_copy(x_hbm.at[idxb[b]], rows[b], sg[b]).wait()
            pltpu.make_async_copy(up_hbm.at[pl.ds(0, C)], upb[b], su[b]).wait()

            @plsc.parallel_loop(0, C, step=1, unroll=4)
            def _add_row(r):
                for l in range(DIM // LANES):
                    sl = pl.ds(l * LANES, LANES)
                    plsc.addupdate(rows[b].at[r, sl], upb[b][r, sl])

            pltpu.async_copy(rows[b], out_hbm.at[pl.ds(start_of(j), C)], so[b])

    prefetch_idx(0, 0)
    prefetch_idx(1, 1)
    issue_loads(0, 0)

    def triple_body(i, _):
        for bb in range(NBUF):
            j = 1 + NBUF * i + bb
            b = (1 + bb) % NBUF
            issue_loads(j, b)
            prefetch_idx(j + 1, (b + 1) % NBUF)
            compute_write(j - 1, bb)
        return 0

    lax.fori_loop(0, J_MAX // NBUF, triple_body, 0)

    for b in range(NBUF):
        pltpu.make_async_copy(rows[b], out_hbm.at[pl.ds(0, C)], so[b]).wait()

    if TAIL:
        @pl.when(wid == 0)
        def _():
            pltpu.sync_copy(inv_hbm.at[pl.ds(TAIL_START, TAIL)],
                            idxb[0].at[pl.ds(0, TAIL)])
            pltpu.async_copy(x_hbm.at[idxb[0].at[pl.ds(0, TAIL)]],
                             rows[0].at[pl.ds(0, TAIL)], sg[0]).wait()
            pltpu.sync_copy(up_hbm.at[pl.ds(TAIL_START, TAIL)],
                            upb[0].at[pl.ds(0, TAIL)])

            @plsc.parallel_loop(0, TAIL, step=1, unroll=4)
            def _add_row_tail(r):
                for l in range(DIM // LANES):
                    sl = pl.ds(l * LANES, LANES)
                    plsc.addupdate(rows[0].at[r, sl], upb[0][r, sl])

            pltpu.sync_copy(rows[0].at[pl.ds(0, TAIL)],
                            out_hbm.at[pl.ds(TAIL_START, TAIL)])


@jax.jit
def _run(x_features, up_x_features, unq_inv):
    mesh = plsc.VectorSubcoreMesh(core_axis_name="c", subcore_axis_name="s")
    return pl.kernel(
        _sc_kernel,
        mesh=mesh,
        out_type=jax.ShapeDtypeStruct((N_FINE, DIM), jnp.float32),
        scratch_types=(
            [pltpu.VMEM((C,), jnp.int32)] * NBUF
            + [pltpu.VMEM((C, DIM), jnp.float32)] * (2 * NBUF)
            + [pltpu.SemaphoreType.DMA] * (4 * NBUF)
        ),
    )(x_features, up_x_features, unq_inv)


def kernel(x_features, up_x_features, unq_inv):
    return _run(x_features, up_x_features, unq_inv.astype(jnp.int32))
